# baseline (speedup 1.0000x reference)
import functools

import jax
import jax.numpy as jnp
from jax import lax
from jax.experimental import pallas as pl
from jax.experimental.pallas import tpu as pltpu
from jax.experimental.pallas import tpu_sc as plsc

T = 4
N = 10000
E = 160000
F = 128
D = 128

NC = 2
NS = 16
EB = 24
EPAD = 162816
NROW = EPAD // EB
RB = NROW // NS
CH = 8
NP = 10240
HALF = NP // NC
ZR = 56
MROWS = N + ZR
MPT = 640

BN = 1000


def _edge_pass(m, src2, dst2, zeros):
    mesh = plsc.VectorSubcoreMesh(core_axis_name="c", subcore_axis_name="s")

    @functools.partial(
        pl.kernel,
        out_type=jax.ShapeDtypeStruct((NP, D), jnp.float32),
        mesh=mesh,
        scratch_types=[
            pltpu.VMEM((CH, EB), jnp.int32),
            pltpu.VMEM((CH, EB), jnp.int32),
            pltpu.VMEM((EB, D), jnp.float32),
            pltpu.VMEM((EB, D), jnp.float32),
            pltpu.VMEM_SHARED((HALF, D), jnp.float32),
            pltpu.VMEM_SHARED((MROWS, D), jnp.float32),
            pltpu.SemaphoreType.DMA,
            pltpu.SemaphoreType.DMA,
            pltpu.SemaphoreType.DMA,
            pltpu.SemaphoreType.DMA,
        ],
    )
    def k(m_hbm, src_hbm, dst_hbm, z_hbm, out_hbm, src_v, dst_v, rows0,
          rows1, acc_sh, m_sh, sem0, sem1, semz, semm):
        cid = lax.axis_index("c")
        sid = lax.axis_index("s")
        zrows = pl.ds(sid * (HALF // NS), HALF // NS)
        pltpu.async_copy(z_hbm.at[zrows], acc_sh.at[zrows], semz)

        @pl.when(sid < NS - 1)
        def _():
            mrows = pl.ds(sid * MPT, MPT)
            pltpu.async_copy(m_hbm.at[mrows], m_sh.at[mrows], semm)

        @pl.when(sid == NS - 1)
        def _():
            mrows = pl.ds((NS - 1) * MPT, N - (NS - 1) * MPT)
            pltpu.async_copy(m_hbm.at[mrows], m_sh.at[mrows], semm)
            pltpu.async_copy(z_hbm.at[pl.ds(0, ZR)],
                             m_sh.at[pl.ds(N, ZR)], semm)

        base = sid * RB
        pltpu.make_async_copy(z_hbm.at[zrows], acc_sh.at[zrows],
                              semz).wait()

        @pl.when(sid < NS - 1)
        def _():
            mrows = pl.ds(sid * MPT, MPT)
            pltpu.make_async_copy(m_hbm.at[mrows], m_sh.at[mrows],
                                  semm).wait()

        @pl.when(sid == NS - 1)
        def _():
            mrows = pl.ds((NS - 1) * MPT, N - (NS - 1) * MPT)
            pltpu.make_async_copy(m_hbm.at[mrows], m_sh.at[mrows],
                                  semm).wait()
            pltpu.make_async_copy(z_hbm.at[pl.ds(0, ZR)],
                                  m_sh.at[pl.ds(N, ZR)], semm).wait()

        plsc.subcore_barrier()

        @pl.loop(0, RB, step=CH)
        def _(r):
            pltpu.sync_copy(src_hbm.at[cid].at[pl.ds(base + r, CH)], src_v)
            pltpu.sync_copy(dst_hbm.at[cid].at[pl.ds(base + r, CH)], dst_v)
            pltpu.async_copy(m_sh.at[src_v.at[0]], rows0, sem0)
            pltpu.async_copy(m_sh.at[src_v.at[1]], rows1, sem1)

            @pl.loop(0, CH - 2, step=2)
            def _(q):
                pltpu.make_async_copy(m_sh.at[src_v.at[q]], rows0,
                                      sem0).wait()
                pltpu.sync_copy(rows0, acc_sh.at[dst_v.at[q]], add=True)
                pltpu.async_copy(m_sh.at[src_v.at[q + 2]], rows0, sem0)
                pltpu.make_async_copy(m_sh.at[src_v.at[q + 1]], rows1,
                                      sem1).wait()
                pltpu.sync_copy(rows1, acc_sh.at[dst_v.at[q + 1]], add=True)
                pltpu.async_copy(m_sh.at[src_v.at[q + 3]], rows1, sem1)

            pltpu.make_async_copy(m_sh.at[src_v.at[CH - 2]], rows0,
                                  sem0).wait()
            pltpu.sync_copy(rows0, acc_sh.at[dst_v.at[CH - 2]], add=True)
            pltpu.make_async_copy(m_sh.at[src_v.at[CH - 1]], rows1,
                                  sem1).wait()
            pltpu.sync_copy(rows1, acc_sh.at[dst_v.at[CH - 1]], add=True)

        plsc.subcore_barrier()
        wrows = pl.ds(sid * (HALF // NS), HALF // NS)
        orows = pl.ds(cid * HALF + sid * (HALF // NS), HALF // NS)
        pltpu.sync_copy(acc_sh.at[wrows], out_hbm.at[orows])

    return k(m, src2, dst2, zeros)


def _gru(agg, h, wih, whh, bih, bhh):
    gi = jnp.dot(agg, wih, preferred_element_type=jnp.float32) + bih
    gh = jnp.dot(h, whh, preferred_element_type=jnp.float32) + bhh
    r = jax.nn.sigmoid(gi[:, :D] + gh[:, :D])
    z = jax.nn.sigmoid(gi[:, D:2 * D] + gh[:, D:2 * D])
    n = jnp.tanh(gi[:, 2 * D:] + r * gh[:, 2 * D:])
    return (1.0 - z) * n + z * h


def _pool_body(x_ref, pw_ref, w0_ref, x_out, m_out):
    xp = jnp.dot(x_ref[...], pw_ref[...], preferred_element_type=jnp.float32)
    x_out[...] = xp
    m_out[...] = jnp.dot(xp, w0_ref[...], preferred_element_type=jnp.float32)


def _gru_m_body(agg_ref, h_ref, wih_ref, whh_ref, bih_ref, bhh_ref, w1_ref,
                h_out, m_out):
    hn = _gru(agg_ref[...], h_ref[...], wih_ref[...], whh_ref[...],
              bih_ref[...], bhh_ref[...])
    h_out[...] = hn
    m_out[...] = jnp.dot(hn, w1_ref[...], preferred_element_type=jnp.float32)


def _gru_lstm_body(agg_ref, h_ref, hs_ref, cs_ref, wih_ref, whh_ref, bih_ref,
                   bhh_ref, lwih_ref, lwhh_ref, lbih_ref, lbhh_ref,
                   h_new, c_new):
    ht = _gru(agg_ref[...], h_ref[...], wih_ref[...], whh_ref[...],
              bih_ref[...], bhh_ref[...])
    hs = hs_ref[...]
    gates = (jnp.dot(ht, lwih_ref[...], preferred_element_type=jnp.float32)
             + lbih_ref[...]
             + jnp.dot(hs, lwhh_ref[...], preferred_element_type=jnp.float32)
             + lbhh_ref[...])
    i = jax.nn.sigmoid(gates[:, :D])
    f = jax.nn.sigmoid(gates[:, D:2 * D])
    g = jnp.tanh(gates[:, 2 * D:3 * D])
    o = jax.nn.sigmoid(gates[:, 3 * D:])
    c2 = f * cs_ref[...] + i * g
    h_new[...] = o * jnp.tanh(c2)
    c_new[...] = c2


def _row_spec(shape=(BN, D)):
    return pl.BlockSpec(shape, lambda i: (i,) + (0,) * (len(shape) - 1))


def _full_spec(shape):
    return pl.BlockSpec(shape, lambda i: (0,) * len(shape))


def _pool_call(xf, pwT, w0):
    grid = (T * N // BN,)
    return pl.pallas_call(
        _pool_body,
        grid=grid,
        in_specs=[_row_spec((BN, F)), _full_spec((F, D)), _full_spec((D, D))],
        out_specs=[_row_spec(), _row_spec()],
        out_shape=[jax.ShapeDtypeStruct((T * N, D), jnp.float32)] * 2,
    )(xf, pwT, w0)


def _gru_m_call(agg, h, wih, whh, bih, bhh, w1):
    grid = (N // BN,)
    return pl.pallas_call(
        _gru_m_body,
        grid=grid,
        in_specs=[
            _row_spec(),
            _row_spec(),
            _full_spec((D, 3 * D)), _full_spec((D, 3 * D)),
            _full_spec((1, 3 * D)), _full_spec((1, 3 * D)),
            _full_spec((D, D)),
        ],
        out_specs=[_row_spec(), _row_spec()],
        out_shape=[jax.ShapeDtypeStruct((N, D), jnp.float32)] * 2,
    )(agg, h, wih, whh, bih, bhh, w1)


def _gru_lstm_call(agg, h, hs, cs, wih, whh, bih, bhh, lwih, lwhh, lbih,
                   lbhh):
    grid = (N // BN,)
    return pl.pallas_call(
        _gru_lstm_body,
        grid=grid,
        in_specs=[
            _row_spec(),
            _row_spec(), _row_spec(), _row_spec(),
            _full_spec((D, 3 * D)), _full_spec((D, 3 * D)),
            _full_spec((1, 3 * D)), _full_spec((1, 3 * D)),
            _full_spec((D, 4 * D)), _full_spec((D, 4 * D)),
            _full_spec((1, 4 * D)), _full_spec((1, 4 * D)),
        ],
        out_specs=[_row_spec(), _row_spec()],
        out_shape=[jax.ShapeDtypeStruct((N, D), jnp.float32)] * 2,
    )(agg, h, hs, cs, wih, whh, bih, bhh, lwih, lwhh, lbih, lbhh)


def kernel(x, edge_index, pool_w, ggc_w, gru_w_ih, gru_w_hh, gru_b_ih,
           gru_b_hh, lstm_w_ih, lstm_w_hh, lstm_b_ih, lstm_b_hh):
    xf = x.reshape(T * N, F)
    xp, m0 = _pool_call(xf, pool_w.T, ggc_w[0])
    xp = xp.reshape(T, N, D)
    m0 = m0.reshape(T, N, D)

    wih = gru_w_ih.T
    whh = gru_w_hh.T
    bih = gru_b_ih.reshape(1, 3 * D)
    bhh = gru_b_hh.reshape(1, 3 * D)
    lwih = lstm_w_ih.T
    lwhh = lstm_w_hh.T
    lbih = lstm_b_ih.reshape(1, 4 * D)
    lbhh = lstm_b_hh.reshape(1, 4 * D)

    zeros = jnp.zeros((HALF, D), jnp.float32)
    hs = jnp.zeros((N, D), jnp.float32)
    cs = hs
    pad = EPAD - E
    epos = jnp.arange(EPAD, dtype=jnp.int32)
    zsrc = N + (epos % ZR)
    zdst = epos % HALF
    outs = []
    for t in range(T):
        dst_s, src_s = jax.lax.sort_key_val(edge_index[t, 1],
                                            edge_index[t, 0])
        srcp = jnp.concatenate([src_s, jnp.full((pad,), -1, jnp.int32)])
        dstp = jnp.concatenate([dst_s, jnp.full((pad,), -1, jnp.int32)])
        src2, dst2 = [], []
        for c in range(NC):
            local = dstp - c * HALF
            ok = (local >= 0) & (local < HALF)
            src2.append(jnp.where(ok, srcp, zsrc))
            dst2.append(jnp.where(ok, local, zdst))
        src2 = jnp.stack(src2).reshape(NC, NROW, EB)
        dst2 = jnp.stack(dst2).reshape(NC, NROW, EB)
        agg = _edge_pass(m0[t], src2, dst2, zeros)
        h1, m1 = _gru_m_call(agg, xp[t], wih, whh, bih, bhh, ggc_w[1])
        agg = _edge_pass(m1, src2, dst2, zeros)
        hs, cs = _gru_lstm_call(agg, h1, hs, cs, wih, whh, bih, bhh,
                                lwih, lwhh, lbih, lbhh)
        outs.append(hs)
    return jnp.stack(outs, axis=1)


# speedup vs baseline: 1.2454x; 1.2454x over previous
import functools

import jax
import jax.numpy as jnp
from jax import lax
from jax.experimental import pallas as pl
from jax.experimental.pallas import tpu as pltpu
from jax.experimental.pallas import tpu_sc as plsc

T = 4
N = 10000
E = 160000
F = 128
D = 128

NC = 2
NS = 16
EB = 32
EPAD = 163840
NROW = EPAD // EB
RB = NROW // NS
CH = 8
NP = 10240
HALF = NP // NC
ZR = 56
MROWS = N + ZR
MPT = 640

BN = 1000


def _edge_pass(m, src2, dst2, zeros):
    mesh = plsc.VectorSubcoreMesh(core_axis_name="c", subcore_axis_name="s")

    @functools.partial(
        pl.kernel,
        out_type=jax.ShapeDtypeStruct((NP, D), jnp.float32),
        mesh=mesh,
        scratch_types=[
            pltpu.VMEM((CH * EB,), jnp.int32),
            pltpu.VMEM((CH, EB), jnp.int32),
            pltpu.VMEM((EB, D), jnp.float32),
            pltpu.VMEM((EB, D), jnp.float32),
            pltpu.VMEM_SHARED((HALF, D), jnp.float32),
            pltpu.VMEM_SHARED((MROWS, D), jnp.float32),
            pltpu.SemaphoreType.DMA,
            pltpu.SemaphoreType.DMA,
            pltpu.SemaphoreType.DMA,
            pltpu.SemaphoreType.DMA,
        ],
    )
    def k(m_hbm, src_hbm, dst_hbm, z_hbm, out_hbm, src_v, dst_v, rows0,
          rows1, acc_sh, m_sh, sem0, sem1, semz, semm):
        cid = lax.axis_index("c")
        sid = lax.axis_index("s")
        zrows = pl.ds(sid * (HALF // NS), HALF // NS)
        pltpu.async_copy(z_hbm.at[zrows], acc_sh.at[zrows], semz)

        @pl.when(sid < NS - 1)
        def _():
            mrows = pl.ds(sid * MPT, MPT)
            pltpu.async_copy(m_hbm.at[mrows], m_sh.at[mrows], semm)

        @pl.when(sid == NS - 1)
        def _():
            mrows = pl.ds((NS - 1) * MPT, N - (NS - 1) * MPT)
            pltpu.async_copy(m_hbm.at[mrows], m_sh.at[mrows], semm)
            pltpu.async_copy(z_hbm.at[pl.ds(0, ZR)],
                             m_sh.at[pl.ds(N, ZR)], semm)

        base = sid * RB
        pltpu.make_async_copy(z_hbm.at[zrows], acc_sh.at[zrows],
                              semz).wait()

        @pl.when(sid < NS - 1)
        def _():
            mrows = pl.ds(sid * MPT, MPT)
            pltpu.make_async_copy(m_hbm.at[mrows], m_sh.at[mrows],
                                  semm).wait()

        @pl.when(sid == NS - 1)
        def _():
            mrows = pl.ds((NS - 1) * MPT, N - (NS - 1) * MPT)
            pltpu.make_async_copy(m_hbm.at[mrows], m_sh.at[mrows],
                                  semm).wait()
            pltpu.make_async_copy(z_hbm.at[pl.ds(0, ZR)],
                                  m_sh.at[pl.ds(N, ZR)], semm).wait()

        plsc.subcore_barrier()

        @pl.loop(0, RB, step=CH)
        def _(r):
            pltpu.sync_copy(
                src_hbm.at[cid].at[pl.ds((base + r) * EB, CH * EB)], src_v)
            pltpu.sync_copy(dst_hbm.at[cid].at[pl.ds(base + r, CH)], dst_v)
            pltpu.async_copy(m_sh.at[src_v.at[pl.ds(0, EB)]], rows0, sem0)
            pltpu.async_copy(m_sh.at[src_v.at[pl.ds(EB, EB)]], rows1, sem1)

            @pl.loop(0, CH - 2, step=2)
            def _(q):
                pltpu.make_async_copy(m_sh.at[src_v.at[pl.ds(q * EB, EB)]],
                                      rows0, sem0).wait()
                pltpu.sync_copy(rows0, acc_sh.at[dst_v.at[q]], add=True)
                pltpu.async_copy(m_sh.at[src_v.at[pl.ds((q + 2) * EB, EB)]],
                                 rows0, sem0)
                pltpu.make_async_copy(
                    m_sh.at[src_v.at[pl.ds((q + 1) * EB, EB)]], rows1,
                    sem1).wait()
                pltpu.sync_copy(rows1, acc_sh.at[dst_v.at[q + 1]], add=True)
                pltpu.async_copy(m_sh.at[src_v.at[pl.ds((q + 3) * EB, EB)]],
                                 rows1, sem1)

            pltpu.make_async_copy(
                m_sh.at[src_v.at[pl.ds((CH - 2) * EB, EB)]], rows0,
                sem0).wait()
            pltpu.sync_copy(rows0, acc_sh.at[dst_v.at[CH - 2]], add=True)
            pltpu.make_async_copy(
                m_sh.at[src_v.at[pl.ds((CH - 1) * EB, EB)]], rows1,
                sem1).wait()
            pltpu.sync_copy(rows1, acc_sh.at[dst_v.at[CH - 1]], add=True)

        plsc.subcore_barrier()
        wrows = pl.ds(sid * (HALF // NS), HALF // NS)
        orows = pl.ds(cid * HALF + sid * (HALF // NS), HALF // NS)
        pltpu.sync_copy(acc_sh.at[wrows], out_hbm.at[orows])

    return k(m, src2, dst2, zeros)


def _gru(agg, h, wih, whh, bih, bhh):
    gi = jnp.dot(agg, wih, preferred_element_type=jnp.float32) + bih
    gh = jnp.dot(h, whh, preferred_element_type=jnp.float32) + bhh
    r = jax.nn.sigmoid(gi[:, :D] + gh[:, :D])
    z = jax.nn.sigmoid(gi[:, D:2 * D] + gh[:, D:2 * D])
    n = jnp.tanh(gi[:, 2 * D:] + r * gh[:, 2 * D:])
    return (1.0 - z) * n + z * h


def _pool_body(x_ref, pw_ref, w0_ref, x_out, m_out):
    xp = jnp.dot(x_ref[...], pw_ref[...], preferred_element_type=jnp.float32)
    x_out[...] = xp
    m_out[...] = jnp.dot(xp, w0_ref[...], preferred_element_type=jnp.float32)


def _gru_m_body(agg_ref, h_ref, wih_ref, whh_ref, bih_ref, bhh_ref, w1_ref,
                h_out, m_out):
    hn = _gru(agg_ref[...], h_ref[...], wih_ref[...], whh_ref[...],
              bih_ref[...], bhh_ref[...])
    h_out[...] = hn
    m_out[...] = jnp.dot(hn, w1_ref[...], preferred_element_type=jnp.float32)


def _gru_lstm_body(agg_ref, h_ref, hs_ref, cs_ref, wih_ref, whh_ref, bih_ref,
                   bhh_ref, lwih_ref, lwhh_ref, lbih_ref, lbhh_ref,
                   h_new, c_new):
    ht = _gru(agg_ref[...], h_ref[...], wih_ref[...], whh_ref[...],
              bih_ref[...], bhh_ref[...])
    hs = hs_ref[...]
    gates = (jnp.dot(ht, lwih_ref[...], preferred_element_type=jnp.float32)
             + lbih_ref[...]
             + jnp.dot(hs, lwhh_ref[...], preferred_element_type=jnp.float32)
             + lbhh_ref[...])
    i = jax.nn.sigmoid(gates[:, :D])
    f = jax.nn.sigmoid(gates[:, D:2 * D])
    g = jnp.tanh(gates[:, 2 * D:3 * D])
    o = jax.nn.sigmoid(gates[:, 3 * D:])
    c2 = f * cs_ref[...] + i * g
    h_new[...] = o * jnp.tanh(c2)
    c_new[...] = c2


def _row_spec(shape=(BN, D)):
    return pl.BlockSpec(shape, lambda i: (i,) + (0,) * (len(shape) - 1))


def _full_spec(shape):
    return pl.BlockSpec(shape, lambda i: (0,) * len(shape))


def _pool_call(xf, pwT, w0):
    grid = (T * N // BN,)
    return pl.pallas_call(
        _pool_body,
        grid=grid,
        in_specs=[_row_spec((BN, F)), _full_spec((F, D)), _full_spec((D, D))],
        out_specs=[_row_spec(), _row_spec()],
        out_shape=[jax.ShapeDtypeStruct((T * N, D), jnp.float32)] * 2,
    )(xf, pwT, w0)


def _gru_m_call(agg, h, wih, whh, bih, bhh, w1):
    grid = (N // BN,)
    return pl.pallas_call(
        _gru_m_body,
        grid=grid,
        in_specs=[
            _row_spec(),
            _row_spec(),
            _full_spec((D, 3 * D)), _full_spec((D, 3 * D)),
            _full_spec((1, 3 * D)), _full_spec((1, 3 * D)),
            _full_spec((D, D)),
        ],
        out_specs=[_row_spec(), _row_spec()],
        out_shape=[jax.ShapeDtypeStruct((N, D), jnp.float32)] * 2,
    )(agg, h, wih, whh, bih, bhh, w1)


def _gru_lstm_call(agg, h, hs, cs, wih, whh, bih, bhh, lwih, lwhh, lbih,
                   lbhh):
    grid = (N // BN,)
    return pl.pallas_call(
        _gru_lstm_body,
        grid=grid,
        in_specs=[
            _row_spec(),
            _row_spec(), _row_spec(), _row_spec(),
            _full_spec((D, 3 * D)), _full_spec((D, 3 * D)),
            _full_spec((1, 3 * D)), _full_spec((1, 3 * D)),
            _full_spec((D, 4 * D)), _full_spec((D, 4 * D)),
            _full_spec((1, 4 * D)), _full_spec((1, 4 * D)),
        ],
        out_specs=[_row_spec(), _row_spec()],
        out_shape=[jax.ShapeDtypeStruct((N, D), jnp.float32)] * 2,
    )(agg, h, hs, cs, wih, whh, bih, bhh, lwih, lwhh, lbih, lbhh)


def kernel(x, edge_index, pool_w, ggc_w, gru_w_ih, gru_w_hh, gru_b_ih,
           gru_b_hh, lstm_w_ih, lstm_w_hh, lstm_b_ih, lstm_b_hh):
    xf = x.reshape(T * N, F)
    xp, m0 = _pool_call(xf, pool_w.T, ggc_w[0])
    xp = xp.reshape(T, N, D)
    m0 = m0.reshape(T, N, D)

    wih = gru_w_ih.T
    whh = gru_w_hh.T
    bih = gru_b_ih.reshape(1, 3 * D)
    bhh = gru_b_hh.reshape(1, 3 * D)
    lwih = lstm_w_ih.T
    lwhh = lstm_w_hh.T
    lbih = lstm_b_ih.reshape(1, 4 * D)
    lbhh = lstm_b_hh.reshape(1, 4 * D)

    zeros = jnp.zeros((HALF, D), jnp.float32)
    hs = jnp.zeros((N, D), jnp.float32)
    cs = hs
    pad = EPAD - E
    epos = jnp.arange(EPAD, dtype=jnp.int32)
    zsrc = N + (epos % ZR)
    zdst = epos % HALF
    outs = []
    for t in range(T):
        srcp = jnp.concatenate(
            [edge_index[t, 0], jnp.full((pad,), -1, jnp.int32)]
        )
        dstp = jnp.concatenate(
            [edge_index[t, 1], jnp.full((pad,), -1, jnp.int32)]
        )
        src2, dst2 = [], []
        for c in range(NC):
            local = dstp - c * HALF
            ok = (local >= 0) & (local < HALF)
            src2.append(jnp.where(ok, srcp, zsrc))
            dst2.append(jnp.where(ok, local, zdst))
        src2 = jnp.stack(src2)
        dst2 = jnp.stack(dst2).reshape(NC, NROW, EB)
        agg = _edge_pass(m0[t], src2, dst2, zeros)
        h1, m1 = _gru_m_call(agg, xp[t], wih, whh, bih, bhh, ggc_w[1])
        agg = _edge_pass(m1, src2, dst2, zeros)
        hs, cs = _gru_lstm_call(agg, h1, hs, cs, wih, whh, bih, bhh,
                                lwih, lwhh, lbih, lbhh)
        outs.append(hs)
    return jnp.stack(outs, axis=1)
